# Initial kernel scaffold; baseline (speedup 1.0000x reference)
import jax
import jax.numpy as jnp
from jax.experimental import pallas as pl


def kernel(mode, speed, hour, congestion, mode_embed, emission_base, energy_base, W1, b1, W2, b2):
    raise NotImplementedError("write your pallas kernel here")


# speedup vs baseline: 17.3796x; 17.3796x over previous
import functools

import jax
import jax.numpy as jnp
from jax import lax
from jax.experimental import pallas as pl
from jax.experimental.pallas import tpu as pltpu
from jax.experimental.pallas import tpu_sc as plsc

NUM_MODES = 6
HID = 32


_CHUNK = 128
_K = 8
_RING = 4


def _sc_emb_body(nc, groups, table_hbm, mode2d_hbm, out_hbm,
                 tbl_s, idx_b, rows, gsem, osem):
    cid = lax.axis_index("c")
    sid = lax.axis_index("s")
    wid = sid * nc + cid

    @pl.when(sid == 0)
    def _():
        pltpu.sync_copy(table_hbm, tbl_s)

    plsc.subcore_barrier()

    row0 = wid * groups * _K

    def drain_one(slot):
        pltpu.make_async_copy(
            out_hbm.at[pl.ds(pl.multiple_of(row0 * _CHUNK, _CHUNK), _CHUNK)],
            rows.at[slot], osem,
        ).wait()

    def group(g, carry):
        crow = pl.multiple_of(row0 + g * _K, _K)
        pltpu.sync_copy(mode2d_hbm.at[pl.ds(crow, _K)], idx_b)
        for j in range(_K):
            s = j % _RING
            if j < _RING:
                @pl.when(g > 0)
                def _():
                    drain_one(s)
            else:
                drain_one(s)
            pltpu.async_copy(tbl_s.at[idx_b.at[j]], rows.at[s], gsem).wait()
            off = pl.multiple_of((crow + j) * _CHUNK, _CHUNK)
            pltpu.async_copy(rows.at[s], out_hbm.at[pl.ds(off, _CHUNK)], osem)
        return carry

    lax.fori_loop(0, groups, group, 0)
    for s in range(_RING):
        drain_one(s)


def _sc_emb(mode2d, table):
    rows_total, chunk = mode2d.shape
    assert chunk == _CHUNK
    n = rows_total * _CHUNK
    d = table.shape[1]
    info = plsc.get_sparse_core_info()
    nw = info.num_cores * info.num_subcores
    assert rows_total % (nw * _K) == 0
    groups = rows_total // (nw * _K)

    mesh = plsc.VectorSubcoreMesh(core_axis_name="c", subcore_axis_name="s")
    kern = functools.partial(
        pl.kernel,
        mesh=mesh,
        out_type=jax.ShapeDtypeStruct((n, d), jnp.float32),
        scratch_types=[
            pltpu.VMEM_SHARED((NUM_MODES, d), jnp.float32),
            pltpu.VMEM((_K, _CHUNK), jnp.int32),
            pltpu.VMEM((_RING, _CHUNK, d), jnp.float32),
            pltpu.SemaphoreType.DMA,
            pltpu.SemaphoreType.DMA,
        ],
    )(functools.partial(_sc_emb_body, info.num_cores, groups))
    return kern(table, mode2d)


_BB = 512

_INV_SQRT2 = 2.0 ** -0.5


def _tc_rates_body(mode_ref, sp_ref, hr_ref, cg_ref,
                   eb_ref, enb_ref, w1_ref, b1_ref, w2_ref, b2_ref,
                   em_ref, en_ref):
    m = mode_ref[...]
    sp = sp_ref[...]
    hr = hr_ref[...] * (_INV_SQRT2 / 24.0)
    cg = cg_ref[...]

    acc_e = jnp.full_like(sp, b2_ref[0])
    acc_n = jnp.full_like(sp, b2_ref[1])
    for j in range(HID):
        y = (sp * (w1_ref[0, j] * _INV_SQRT2) + hr * w1_ref[1, j]
             + cg * (w1_ref[2, j] * _INV_SQRT2) + b1_ref[j] * _INV_SQRT2)
        t = y * (1.0 + lax.erf(y))
        acc_e = acc_e + t * (w2_ref[j, 0] * _INV_SQRT2)
        acc_n = acc_n + t * (w2_ref[j, 1] * _INV_SQRT2)

    base_e = jnp.zeros_like(sp)
    base_n = jnp.zeros_like(sp)
    for k in range(NUM_MODES):
        mk = m == k
        base_e = jnp.where(mk, eb_ref[k], base_e)
        base_n = jnp.where(mk, enb_ref[k], base_n)

    em_ref[...] = jnp.maximum(base_e + acc_e, 0.0)
    en_ref[...] = jnp.maximum(base_n + acc_n, 0.0)


def _tc_rates(mode, speed, hour, congestion,
              emission_base, energy_base, W1, b1, W2, b2):
    b, l = mode.shape
    assert b % _BB == 0
    grid = (b // _BB,)
    big = pl.BlockSpec((_BB, l), lambda i: (i, 0))
    smem = pl.BlockSpec(memory_space=pltpu.SMEM)
    out_shape = jax.ShapeDtypeStruct((b, l), jnp.float32)
    return pl.pallas_call(
        _tc_rates_body,
        grid=grid,
        in_specs=[big, big, big, big, smem, smem, smem, smem, smem, smem],
        out_specs=[big, big],
        out_shape=[out_shape, out_shape],
    )(mode, speed, hour, congestion,
      emission_base, energy_base, W1, b1, W2, b2)


def kernel(mode, speed, hour, congestion, mode_embed, emission_base, energy_base, W1, b1, W2, b2):
    b, l = mode.shape
    d = mode_embed.shape[1]
    n = b * l

    emb = _sc_emb(mode.reshape(n // _CHUNK, _CHUNK), mode_embed)
    em, en = _tc_rates(mode, speed, hour, congestion,
                       emission_base, energy_base, W1, b1, W2, b2)
    return emb.reshape(b, l, d), em, en


# speedup vs baseline: 17.6372x; 1.0148x over previous
import functools

import jax
import jax.numpy as jnp
from jax import lax
from jax.experimental import pallas as pl
from jax.experimental.pallas import tpu as pltpu
from jax.experimental.pallas import tpu_sc as plsc

NUM_MODES = 6
HID = 32


_CHUNK = 128
_K = 16
_RING = 4


def _sc_emb_body(nc, groups, table_hbm, mode2d_hbm, out_hbm,
                 tbl_s, idx_b, rows, gsem, osem):
    cid = lax.axis_index("c")
    sid = lax.axis_index("s")
    wid = sid * nc + cid

    @pl.when(sid == 0)
    def _():
        pltpu.sync_copy(table_hbm, tbl_s)

    plsc.subcore_barrier()

    row0 = wid * groups * _K

    def drain_one(slot):
        pltpu.make_async_copy(
            out_hbm.at[pl.ds(pl.multiple_of(row0 * _CHUNK, _CHUNK), _CHUNK)],
            rows.at[slot], osem,
        ).wait()

    def group(g, carry):
        crow = pl.multiple_of(row0 + g * _K, _K)
        pltpu.sync_copy(mode2d_hbm.at[pl.ds(crow, _K)], idx_b)
        for j in range(_K):
            s = j % _RING
            if j < _RING:
                @pl.when(g > 0)
                def _():
                    drain_one(s)
            else:
                drain_one(s)
            pltpu.async_copy(tbl_s.at[idx_b.at[j]], rows.at[s], gsem).wait()
            off = pl.multiple_of((crow + j) * _CHUNK, _CHUNK)
            pltpu.async_copy(rows.at[s], out_hbm.at[pl.ds(off, _CHUNK)], osem)
        return carry

    lax.fori_loop(0, groups, group, 0)
    for s in range(_RING):
        drain_one(s)


def _sc_emb(mode2d, table):
    rows_total, chunk = mode2d.shape
    assert chunk == _CHUNK
    n = rows_total * _CHUNK
    d = table.shape[1]
    info = plsc.get_sparse_core_info()
    nw = info.num_cores * info.num_subcores
    assert rows_total % (nw * _K) == 0
    groups = rows_total // (nw * _K)

    mesh = plsc.VectorSubcoreMesh(core_axis_name="c", subcore_axis_name="s")
    kern = functools.partial(
        pl.kernel,
        mesh=mesh,
        out_type=jax.ShapeDtypeStruct((n, d), jnp.float32),
        scratch_types=[
            pltpu.VMEM_SHARED((NUM_MODES, d), jnp.float32),
            pltpu.VMEM((_K, _CHUNK), jnp.int32),
            pltpu.VMEM((_RING, _CHUNK, d), jnp.float32),
            pltpu.SemaphoreType.DMA,
            pltpu.SemaphoreType.DMA,
        ],
    )(functools.partial(_sc_emb_body, info.num_cores, groups))
    return kern(table, mode2d)


_BB = 512

_INV_SQRT2 = 2.0 ** -0.5


def _tc_rates_body(mode_ref, sp_ref, hr_ref, cg_ref,
                   eb_ref, enb_ref, w1_ref, b1_ref, w2_ref, b2_ref,
                   em_ref, en_ref):
    m = mode_ref[...]
    sp = sp_ref[...]
    hr = hr_ref[...] * (_INV_SQRT2 / 24.0)
    cg = cg_ref[...]

    acc_e = jnp.full_like(sp, b2_ref[0])
    acc_n = jnp.full_like(sp, b2_ref[1])
    for j in range(HID):
        y = (sp * (w1_ref[0, j] * _INV_SQRT2) + hr * w1_ref[1, j]
             + cg * (w1_ref[2, j] * _INV_SQRT2) + b1_ref[j] * _INV_SQRT2)
        t = y * (1.0 + lax.erf(y))
        acc_e = acc_e + t * (w2_ref[j, 0] * _INV_SQRT2)
        acc_n = acc_n + t * (w2_ref[j, 1] * _INV_SQRT2)

    base_e = jnp.zeros_like(sp)
    base_n = jnp.zeros_like(sp)
    for k in range(NUM_MODES):
        mk = m == k
        base_e = jnp.where(mk, eb_ref[k], base_e)
        base_n = jnp.where(mk, enb_ref[k], base_n)

    em_ref[...] = jnp.maximum(base_e + acc_e, 0.0)
    en_ref[...] = jnp.maximum(base_n + acc_n, 0.0)


def _tc_rates(mode, speed, hour, congestion,
              emission_base, energy_base, W1, b1, W2, b2):
    b, l = mode.shape
    assert b % _BB == 0
    grid = (b // _BB,)
    big = pl.BlockSpec((_BB, l), lambda i: (i, 0))
    smem = pl.BlockSpec(memory_space=pltpu.SMEM)
    out_shape = jax.ShapeDtypeStruct((b, l), jnp.float32)
    return pl.pallas_call(
        _tc_rates_body,
        grid=grid,
        in_specs=[big, big, big, big, smem, smem, smem, smem, smem, smem],
        out_specs=[big, big],
        out_shape=[out_shape, out_shape],
    )(mode, speed, hour, congestion,
      emission_base, energy_base, W1, b1, W2, b2)


def kernel(mode, speed, hour, congestion, mode_embed, emission_base, energy_base, W1, b1, W2, b2):
    b, l = mode.shape
    d = mode_embed.shape[1]
    n = b * l

    emb = _sc_emb(mode.reshape(n // _CHUNK, _CHUNK), mode_embed)
    em, en = _tc_rates(mode, speed, hour, congestion,
                       emission_base, energy_base, W1, b1, W2, b2)
    return emb.reshape(b, l, d), em, en
